# Initial kernel scaffold; baseline (speedup 1.0000x reference)
import jax
import jax.numpy as jnp
from jax.experimental import pallas as pl


def kernel(z, charge, energy_table):
    raise NotImplementedError("write your pallas kernel here")


# speedup vs baseline: 464.4495x; 464.4495x over previous
import functools

import jax
import jax.numpy as jnp
from jax import lax
from jax.experimental import pallas as pl
from jax.experimental.pallas import tpu as pltpu
from jax.experimental.pallas import tpu_sc as plsc

_LANES = 16


def _sc_lookup(table_pad, z, charge):
    n = z.shape[0]
    info = plsc.get_sparse_core_info()
    nw = info.num_cores * info.num_subcores
    per_w = n // nw
    tpad = table_pad.shape[0]
    mesh = plsc.VectorSubcoreMesh(core_axis_name="c", subcore_axis_name="s")

    @functools.partial(
        pl.kernel,
        mesh=mesh,
        out_type=jax.ShapeDtypeStruct((n,), jnp.float32),
        compiler_params=pltpu.CompilerParams(needs_layout_passes=False),
        scratch_types=[
            pltpu.VMEM((tpad,), jnp.float32),
            pltpu.VMEM((per_w,), jnp.int32),
            pltpu.VMEM((per_w,), jnp.int32),
            pltpu.VMEM((per_w,), jnp.float32),
            pltpu.SemaphoreType.DMA,
            pltpu.SemaphoreType.DMA,
        ],
    )
    def k(table_hbm, z_hbm, q_hbm, out_hbm, t_v, z_v, q_v, o_v, sem_z, sem_q):
        wid = lax.axis_index("s") * info.num_cores + lax.axis_index("c")
        base = wid * per_w
        cp_z = pltpu.async_copy(z_hbm.at[pl.ds(base, per_w)], z_v, sem_z)
        cp_q = pltpu.async_copy(q_hbm.at[pl.ds(base, per_w)], q_v, sem_q)
        pltpu.sync_copy(table_hbm, t_v)
        cp_z.wait()
        cp_q.wait()

        def body(i, _):
            off = i * _LANES
            z16 = z_v[pl.ds(off, _LANES)]
            q16 = q_v[pl.ds(off, _LANES)]
            idx = z16 * 3 + q16
            o_v[pl.ds(off, _LANES)] = plsc.load_gather(t_v, [idx])
            return 0

        lax.fori_loop(0, per_w // _LANES, body, 0)
        pltpu.sync_copy(o_v, out_hbm.at[pl.ds(base, per_w)])

    return k(table_pad, z, charge)


def kernel(z, charge, energy_table):
    table_pad = jnp.zeros((128,), jnp.float32).at[:108].set(
        energy_table.reshape(-1)
    )
    return _sc_lookup(table_pad, z, charge)


# speedup vs baseline: 534.2434x; 1.1503x over previous
import functools

import jax
import jax.numpy as jnp
from jax import lax
from jax.experimental import pallas as pl
from jax.experimental.pallas import tpu as pltpu
from jax.experimental.pallas import tpu_sc as plsc

_LANES = 16


def _sc_lookup(table_pad, z, charge):
    n = z.shape[0]
    info = plsc.get_sparse_core_info()
    nw = info.num_cores * info.num_subcores
    per_w = n // nw
    tpad = table_pad.shape[0]
    mesh = plsc.VectorSubcoreMesh(core_axis_name="c", subcore_axis_name="s")

    @functools.partial(
        pl.kernel,
        mesh=mesh,
        out_type=jax.ShapeDtypeStruct((n,), jnp.float32),
        compiler_params=pltpu.CompilerParams(needs_layout_passes=False),
        scratch_types=[
            pltpu.VMEM((tpad,), jnp.float32),
            pltpu.VMEM((per_w,), jnp.int32),
            pltpu.VMEM((per_w,), jnp.int32),
            pltpu.VMEM((per_w,), jnp.float32),
            pltpu.SemaphoreType.DMA,
            pltpu.SemaphoreType.DMA,
        ],
    )
    def k(table_hbm, z_hbm, q_hbm, out_hbm, t_v, z_v, q_v, o_v, sem_z, sem_q):
        wid = lax.axis_index("s") * info.num_cores + lax.axis_index("c")
        base = wid * per_w
        cp_z = pltpu.async_copy(z_hbm.at[pl.ds(base, per_w)], z_v, sem_z)
        cp_q = pltpu.async_copy(q_hbm.at[pl.ds(base, per_w)], q_v, sem_q)
        pltpu.sync_copy(table_hbm, t_v)
        cp_z.wait()
        cp_q.wait()

        @plsc.parallel_loop(0, per_w, _LANES, unroll=8)
        def body(off):
            z16 = z_v[pl.ds(off, _LANES)]
            q16 = q_v[pl.ds(off, _LANES)]
            idx = z16 * 3 + q16
            o_v[pl.ds(off, _LANES)] = plsc.load_gather(t_v, [idx])
        pltpu.sync_copy(o_v, out_hbm.at[pl.ds(base, per_w)])

    return k(table_pad, z, charge)


def kernel(z, charge, energy_table):
    table_pad = jnp.zeros((128,), jnp.float32).at[:108].set(
        energy_table.reshape(-1)
    )
    return _sc_lookup(table_pad, z, charge)
